# baseline (speedup 1.0000x reference)
import functools

import jax
import jax.numpy as jnp
from jax.experimental import pallas as pl
from jax.experimental.pallas import tpu as pltpu
from jax.experimental.pallas import tpu_sc as plsc

_B, _N, _N_UP, _C, _K, _DIM = 2, 4096, 8192, 128, 3, 128
_TILE_Q = 256
_GW = 128


_RG = 8
_NCHUNK = _N // 128


def _knn_body(q_ref, kT_ref, oi_ref, ow_ref, vals_ref, gidx_ref, qk_ref):
    bi = pl.program_id(0)
    kT = kT_ref[0]
    kx, ky, kz = kT[0:1, :], kT[1:2, :], kT[2:3, :]
    ksq = kx * kx + ky * ky + kz * kz
    q = q_ref[0]
    qk_ref[...] = jax.lax.dot(q.astype(jnp.bfloat16), kT.astype(jnp.bfloat16),
                              preferred_element_type=jnp.float32)

    lane = jax.lax.broadcasted_iota(jnp.int32, (_RG, 128), 1)
    big = jnp.full((_RG, 128), 3e38, jnp.float32)
    zero_i = jnp.zeros((_RG, 128), jnp.int32)

    def row_group(rg, _):
        qg = q_ref[0, pl.ds(rg * _RG, _RG), :]
        qx, qy, qz = qg[:, 0:1], qg[:, 1:2], qg[:, 2:3]
        qsq = qx * qx + qy * qy + qz * qz
        qkg = qk_ref[pl.ds(rg * _RG, _RG), :]

        A1 = A2 = A3 = big
        I1 = I2 = I3 = zero_i
        for c in range(_NCHUNK):
            sl = slice(c * 128, (c + 1) * 128)
            v = (qsq + ksq[:, sl]) - 2.0 * qkg[:, sl]
            lt1 = v < A1
            lt2 = v < A2
            lt3 = v < A3
            A3 = jnp.where(lt2, A2, jnp.where(lt3, v, A3))
            I3 = jnp.where(lt2, I2, jnp.where(lt3, c, I3))
            A2 = jnp.where(lt1, A1, jnp.where(lt2, v, A2))
            I2 = jnp.where(lt1, I1, jnp.where(lt2, c, I2))
            A1 = jnp.where(lt1, v, A1)
            I1 = jnp.where(lt1, c, I1)

        rows = pl.ds(rg * _RG, _RG)
        vals_ref[rows, 0:128] = A1
        vals_ref[rows, 128:256] = A2
        vals_ref[rows, 256:384] = A3
        gidx_ref[rows, 0:128] = I1 * 128 + lane
        gidx_ref[rows, 128:256] = I2 * 128 + lane
        gidx_ref[rows, 256:384] = I3 * 128 + lane

    jax.lax.fori_loop(0, _TILE_Q // _RG, row_group, None)

    V = vals_ref[...]
    G = gidx_ref[...]
    idxs, ws = [], []
    for k in range(_K):
        m = jnp.min(V, axis=1, keepdims=True)
        isel = jnp.min(jnp.where(V == m, G, _N), axis=1, keepdims=True)
        dist = jnp.sqrt(jnp.maximum(m, 1e-12))
        ws.append(1.0 / (dist + 1e-6))
        idxs.append(isel)
        if k < _K - 1:
            V = jnp.where((V == m) & (G == isel), jnp.float32(3e38), V)

    wsum = ws[0] + ws[1] + ws[2]
    ow_ref[0] = jnp.concatenate([w / wsum for w in ws], axis=1)
    oi_ref[0] = jnp.concatenate(idxs, axis=1) + bi * _N


def _tc_knn(pos_up, posT):
    from jax.experimental.pallas import tpu as pltpu_mod
    grid = (_B, _N_UP // _TILE_Q)
    return pl.pallas_call(
        _knn_body,
        grid=grid,
        in_specs=[
            pl.BlockSpec((1, _TILE_Q, 3), lambda bi, qi: (bi, qi, 0)),
            pl.BlockSpec((1, 3, _N), lambda bi, qi: (bi, 0, 0)),
        ],
        out_specs=[
            pl.BlockSpec((1, _TILE_Q, _K), lambda bi, qi: (bi, qi, 0)),
            pl.BlockSpec((1, _TILE_Q, _K), lambda bi, qi: (bi, qi, 0)),
        ],
        out_shape=[
            jax.ShapeDtypeStruct((_B, _N_UP, _K), jnp.int32),
            jax.ShapeDtypeStruct((_B, _N_UP, _K), jnp.float32),
        ],
        scratch_shapes=[
            pltpu_mod.VMEM((_TILE_Q, 3 * 128), jnp.float32),
            pltpu_mod.VMEM((_TILE_Q, 3 * 128), jnp.int32),
            pltpu_mod.VMEM((_TILE_Q, _N), jnp.float32),
        ],
    )(pos_up, posT)


def _sc_gather(feature2, flat_idx):
    num_idx = flat_idx.shape[1]
    mesh = plsc.VectorSubcoreMesh(core_axis_name="core",
                                  subcore_axis_name="subcore")

    @pl.kernel(out_type=jax.ShapeDtypeStruct((num_idx, _C), jnp.float32),
               mesh=mesh)
    def kern(x_hbm, i_hbm, o_hbm):
        def body(i_vmem, o_vmem):
            pltpu.sync_copy(x_hbm.at[i_vmem.at[0]], o_vmem)

        pltpu.emit_pipeline(
            body,
            grid=(num_idx // _GW,),
            in_specs=[pl.BlockSpec((1, _GW), index_map=lambda i: (0, i))],
            out_specs=[pl.BlockSpec((_GW, _C), index_map=lambda i: (i, 0))],
            core_axis_name=("core", "subcore"),
            dimension_semantics=(pltpu.PARALLEL,),
        )(i_hbm, o_hbm)

    return kern(feature2, flat_idx)


def _interp_body(g_ref, w_ref, wm_ref, b_ref, o_ref):
    wn = w_ref[0]
    G = (wn[:, 0:1] * g_ref[0, 0]
         + wn[:, 1:2] * g_ref[1, 0]
         + wn[:, 2:3] * g_ref[2, 0])
    out = jax.lax.dot(G.astype(jnp.bfloat16), wm_ref[...].astype(jnp.bfloat16),
                      preferred_element_type=jnp.float32)
    o_ref[0] = jnp.maximum(out + b_ref[...], 0.0)


def _tc_interp(gathered, weights, W, b2):
    grid = (_B, _N_UP // _TILE_Q)
    return pl.pallas_call(
        _interp_body,
        grid=grid,
        in_specs=[
            pl.BlockSpec((_K, 1, _TILE_Q, _C), lambda bi, qi: (0, bi, qi, 0)),
            pl.BlockSpec((1, _TILE_Q, _K), lambda bi, qi: (bi, qi, 0)),
            pl.BlockSpec((_C, _DIM), lambda bi, qi: (0, 0)),
            pl.BlockSpec((1, _DIM), lambda bi, qi: (0, 0)),
        ],
        out_specs=pl.BlockSpec((1, _TILE_Q, _DIM), lambda bi, qi: (bi, qi, 0)),
        out_shape=jax.ShapeDtypeStruct((_B, _N_UP, _DIM), jnp.float32),
    )(gathered, weights, W, b2)


@jax.jit
def kernel(feature, pos, pos_up, W, b):
    posT = jnp.swapaxes(pos, 1, 2)
    b2 = b.reshape(1, _DIM)
    idx, weights = _tc_knn(pos_up, posT)
    flat_idx = jnp.transpose(idx, (2, 0, 1)).reshape(1, _K * _B * _N_UP)
    feature2 = feature.reshape(_B * _N, _C)
    gathered = _sc_gather(feature2, flat_idx)
    gathered = gathered.reshape(_K, _B, _N_UP, _C)
    return _tc_interp(gathered, weights, W, b2)


# speedup vs baseline: 1.5161x; 1.5161x over previous
import functools

import jax
import jax.numpy as jnp
from jax.experimental import pallas as pl
from jax.experimental.pallas import tpu as pltpu
from jax.experimental.pallas import tpu_sc as plsc

_B, _N, _N_UP, _C, _K, _DIM = 2, 4096, 8192, 128, 3, 128
_TILE_Q = 256
_GW = 128


_RG = 32
_NCHUNK = _N // 128


def _knn_body(q_ref, kT_ref, oi_ref, ow_ref, vals_ref, gidx_ref, qk_ref):
    bi = pl.program_id(0)
    kT = kT_ref[0]
    kx, ky, kz = kT[0:1, :], kT[1:2, :], kT[2:3, :]
    ksq = kx * kx + ky * ky + kz * kz
    q = q_ref[0]
    qk_ref[...] = jax.lax.dot(q.astype(jnp.bfloat16), kT.astype(jnp.bfloat16),
                              preferred_element_type=jnp.float32)

    lane = jax.lax.broadcasted_iota(jnp.int32, (_RG, 128), 1)
    big = jnp.full((_RG, 128), 3e38, jnp.float32)
    zero_i = jnp.zeros((_RG, 128), jnp.int32)

    def row_group(rg, _):
        qg = q_ref[0, pl.ds(rg * _RG, _RG), :]
        qx, qy, qz = qg[:, 0:1], qg[:, 1:2], qg[:, 2:3]
        qsq = qx * qx + qy * qy + qz * qz
        qkg = qk_ref[pl.ds(rg * _RG, _RG), :]

        A1 = A2 = A3 = big
        I1 = I2 = I3 = zero_i
        for c in range(_NCHUNK):
            sl = slice(c * 128, (c + 1) * 128)
            v = (qsq + ksq[:, sl]) - 2.0 * qkg[:, sl]
            lt1 = v < A1
            lt2 = v < A2
            lt3 = v < A3
            A3 = jnp.where(lt2, A2, jnp.where(lt3, v, A3))
            I3 = jnp.where(lt2, I2, jnp.where(lt3, c, I3))
            A2 = jnp.where(lt1, A1, jnp.where(lt2, v, A2))
            I2 = jnp.where(lt1, I1, jnp.where(lt2, c, I2))
            A1 = jnp.where(lt1, v, A1)
            I1 = jnp.where(lt1, c, I1)

        rows = pl.ds(rg * _RG, _RG)
        vals_ref[rows, 0:128] = A1
        vals_ref[rows, 128:256] = A2
        vals_ref[rows, 256:384] = A3
        gidx_ref[rows, 0:128] = I1 * 128 + lane
        gidx_ref[rows, 128:256] = I2 * 128 + lane
        gidx_ref[rows, 256:384] = I3 * 128 + lane

    jax.lax.fori_loop(0, _TILE_Q // _RG, row_group, None)

    V = vals_ref[...]
    G = gidx_ref[...]
    idxs, ws = [], []
    for k in range(_K):
        m = jnp.min(V, axis=1, keepdims=True)
        isel = jnp.min(jnp.where(V == m, G, _N), axis=1, keepdims=True)
        dist = jnp.sqrt(jnp.maximum(m, 1e-12))
        ws.append(1.0 / (dist + 1e-6))
        idxs.append(isel)
        if k < _K - 1:
            V = jnp.where((V == m) & (G == isel), jnp.float32(3e38), V)

    wsum = ws[0] + ws[1] + ws[2]
    ow_ref[0] = jnp.concatenate([w / wsum for w in ws], axis=1)
    oi_ref[0] = jnp.concatenate(idxs, axis=1) + bi * _N


def _tc_knn(pos_up, posT):
    from jax.experimental.pallas import tpu as pltpu_mod
    grid = (_B, _N_UP // _TILE_Q)
    return pl.pallas_call(
        _knn_body,
        grid=grid,
        in_specs=[
            pl.BlockSpec((1, _TILE_Q, 3), lambda bi, qi: (bi, qi, 0)),
            pl.BlockSpec((1, 3, _N), lambda bi, qi: (bi, 0, 0)),
        ],
        out_specs=[
            pl.BlockSpec((1, _TILE_Q, _K), lambda bi, qi: (bi, qi, 0)),
            pl.BlockSpec((1, _TILE_Q, _K), lambda bi, qi: (bi, qi, 0)),
        ],
        out_shape=[
            jax.ShapeDtypeStruct((_B, _N_UP, _K), jnp.int32),
            jax.ShapeDtypeStruct((_B, _N_UP, _K), jnp.float32),
        ],
        scratch_shapes=[
            pltpu_mod.VMEM((_TILE_Q, 3 * 128), jnp.float32),
            pltpu_mod.VMEM((_TILE_Q, 3 * 128), jnp.int32),
            pltpu_mod.VMEM((_TILE_Q, _N), jnp.float32),
        ],
    )(pos_up, posT)


def _sc_gather(feature2, flat_idx):
    num_idx = flat_idx.shape[1]
    mesh = plsc.VectorSubcoreMesh(core_axis_name="core",
                                  subcore_axis_name="subcore")

    @pl.kernel(out_type=jax.ShapeDtypeStruct((num_idx, _C), jnp.float32),
               mesh=mesh)
    def kern(x_hbm, i_hbm, o_hbm):
        def body(i_vmem, o_vmem):
            pltpu.sync_copy(x_hbm.at[i_vmem.at[0]], o_vmem)

        pltpu.emit_pipeline(
            body,
            grid=(num_idx // _GW,),
            in_specs=[pl.BlockSpec((1, _GW), index_map=lambda i: (0, i))],
            out_specs=[pl.BlockSpec((_GW, _C), index_map=lambda i: (i, 0))],
            core_axis_name=("core", "subcore"),
            dimension_semantics=(pltpu.PARALLEL,),
        )(i_hbm, o_hbm)

    return kern(feature2, flat_idx)


def _interp_body(g_ref, w_ref, wm_ref, b_ref, o_ref):
    wn = w_ref[0]
    G = (wn[:, 0:1] * g_ref[0, 0]
         + wn[:, 1:2] * g_ref[1, 0]
         + wn[:, 2:3] * g_ref[2, 0])
    out = jax.lax.dot(G.astype(jnp.bfloat16), wm_ref[...].astype(jnp.bfloat16),
                      preferred_element_type=jnp.float32)
    o_ref[0] = jnp.maximum(out + b_ref[...], 0.0)


def _tc_interp(gathered, weights, W, b2):
    grid = (_B, _N_UP // _TILE_Q)
    return pl.pallas_call(
        _interp_body,
        grid=grid,
        in_specs=[
            pl.BlockSpec((_K, 1, _TILE_Q, _C), lambda bi, qi: (0, bi, qi, 0)),
            pl.BlockSpec((1, _TILE_Q, _K), lambda bi, qi: (bi, qi, 0)),
            pl.BlockSpec((_C, _DIM), lambda bi, qi: (0, 0)),
            pl.BlockSpec((1, _DIM), lambda bi, qi: (0, 0)),
        ],
        out_specs=pl.BlockSpec((1, _TILE_Q, _DIM), lambda bi, qi: (bi, qi, 0)),
        out_shape=jax.ShapeDtypeStruct((_B, _N_UP, _DIM), jnp.float32),
    )(gathered, weights, W, b2)


@jax.jit
def kernel(feature, pos, pos_up, W, b):
    posT = jnp.swapaxes(pos, 1, 2)
    b2 = b.reshape(1, _DIM)
    idx, weights = _tc_knn(pos_up, posT)
    flat_idx = jnp.transpose(idx, (2, 0, 1)).reshape(1, _K * _B * _N_UP)
    feature2 = feature.reshape(_B * _N, _C)
    gathered = _sc_gather(feature2, flat_idx)
    gathered = gathered.reshape(_K, _B, _N_UP, _C)
    return _tc_interp(gathered, weights, W, b2)


# speedup vs baseline: 1.5713x; 1.0364x over previous
import functools

import jax
import jax.numpy as jnp
from jax.experimental import pallas as pl
from jax.experimental.pallas import tpu as pltpu
from jax.experimental.pallas import tpu_sc as plsc

_B, _N, _N_UP, _C, _K, _DIM = 2, 4096, 8192, 128, 3, 128
_TILE_Q = 256
_GW = 128


_RG = 32
_NCHUNK = _N // 128


def _knn_body(q_ref, kT_ref, oi_ref, ow_ref, vals_ref, gidx_ref, qk_ref):
    bi = pl.program_id(0)
    kT = kT_ref[0]
    kx, ky, kz = kT[0:1, :], kT[1:2, :], kT[2:3, :]
    ksq = kx * kx + ky * ky + kz * kz
    q = q_ref[0]
    qk_ref[...] = jax.lax.dot(q.astype(jnp.bfloat16), kT.astype(jnp.bfloat16),
                              preferred_element_type=jnp.float32)

    lane = jax.lax.broadcasted_iota(jnp.int32, (_RG, 128), 1).astype(jnp.float32)
    big = jnp.full((_RG, 128), 3e38, jnp.float32)

    def row_group(rg, _):
        qg = q_ref[0, pl.ds(rg * _RG, _RG), :]
        qx, qy, qz = qg[:, 0:1], qg[:, 1:2], qg[:, 2:3]
        qsq = qx * qx + qy * qy + qz * qz
        qkg = qk_ref[pl.ds(rg * _RG, _RG), :]

        A1 = A2 = A3 = big
        I1 = I2 = I3 = jnp.zeros((_RG, 128), jnp.float32)
        for c in range(_NCHUNK):
            sl = slice(c * 128, (c + 1) * 128)
            v = (qsq + ksq[:, sl]) - 2.0 * qkg[:, sl]
            lt1 = v < A1
            lt2 = v < A2
            lt3 = v < A3
            fc = jnp.float32(c)
            A3 = jnp.minimum(jnp.maximum(v, A2), A3)
            I3 = jnp.where(lt2, I2, jnp.where(lt3, fc, I3))
            A2 = jnp.minimum(jnp.maximum(v, A1), A2)
            I2 = jnp.where(lt1, I1, jnp.where(lt2, fc, I2))
            A1 = jnp.minimum(v, A1)
            I1 = jnp.where(lt1, fc, I1)

        rows = pl.ds(rg * _RG, _RG)
        vals_ref[rows, 0:128] = A1
        vals_ref[rows, 128:256] = A2
        vals_ref[rows, 256:384] = A3
        gidx_ref[rows, 0:128] = I1 * 128.0 + lane
        gidx_ref[rows, 128:256] = I2 * 128.0 + lane
        gidx_ref[rows, 256:384] = I3 * 128.0 + lane

    jax.lax.fori_loop(0, _TILE_Q // _RG, row_group, None)

    V = vals_ref[...]
    G = gidx_ref[...]
    idxs, ws = [], []
    for k in range(_K):
        m = jnp.min(V, axis=1, keepdims=True)
        isel = jnp.min(jnp.where(V == m, G, jnp.float32(_N)),
                       axis=1, keepdims=True)
        dist = jnp.sqrt(jnp.maximum(m, 1e-12))
        ws.append(1.0 / (dist + 1e-6))
        idxs.append(isel)
        if k < _K - 1:
            V = jnp.where((V == m) & (G == isel), jnp.float32(3e38), V)

    wsum = ws[0] + ws[1] + ws[2]
    ow_ref[0] = jnp.concatenate([w / wsum for w in ws], axis=1)
    oi_ref[0] = (jnp.concatenate(idxs, axis=1).astype(jnp.int32)
                 + bi * _N)


def _tc_knn(pos_up, posT):
    from jax.experimental.pallas import tpu as pltpu_mod
    grid = (_B, _N_UP // _TILE_Q)
    return pl.pallas_call(
        _knn_body,
        grid=grid,
        in_specs=[
            pl.BlockSpec((1, _TILE_Q, 3), lambda bi, qi: (bi, qi, 0)),
            pl.BlockSpec((1, 3, _N), lambda bi, qi: (bi, 0, 0)),
        ],
        out_specs=[
            pl.BlockSpec((1, _TILE_Q, _K), lambda bi, qi: (bi, qi, 0)),
            pl.BlockSpec((1, _TILE_Q, _K), lambda bi, qi: (bi, qi, 0)),
        ],
        out_shape=[
            jax.ShapeDtypeStruct((_B, _N_UP, _K), jnp.int32),
            jax.ShapeDtypeStruct((_B, _N_UP, _K), jnp.float32),
        ],
        scratch_shapes=[
            pltpu_mod.VMEM((_TILE_Q, 3 * 128), jnp.float32),
            pltpu_mod.VMEM((_TILE_Q, 3 * 128), jnp.float32),
            pltpu_mod.VMEM((_TILE_Q, _N), jnp.float32),
        ],
    )(pos_up, posT)


def _sc_gather(feature2, flat_idx):
    num_idx = flat_idx.shape[1]
    mesh = plsc.VectorSubcoreMesh(core_axis_name="core",
                                  subcore_axis_name="subcore")

    @pl.kernel(out_type=jax.ShapeDtypeStruct((num_idx, _C), jnp.float32),
               mesh=mesh)
    def kern(x_hbm, i_hbm, o_hbm):
        def body(i_vmem, o_vmem):
            pltpu.sync_copy(x_hbm.at[i_vmem.at[0]], o_vmem)

        pltpu.emit_pipeline(
            body,
            grid=(num_idx // _GW,),
            in_specs=[pl.BlockSpec((1, _GW), index_map=lambda i: (0, i))],
            out_specs=[pl.BlockSpec((_GW, _C), index_map=lambda i: (i, 0))],
            core_axis_name=("core", "subcore"),
            dimension_semantics=(pltpu.PARALLEL,),
        )(i_hbm, o_hbm)

    return kern(feature2, flat_idx)


def _interp_body(g_ref, w_ref, wm_ref, b_ref, o_ref):
    wn = w_ref[0]
    G = (wn[:, 0:1] * g_ref[0, 0]
         + wn[:, 1:2] * g_ref[1, 0]
         + wn[:, 2:3] * g_ref[2, 0])
    out = jax.lax.dot(G.astype(jnp.bfloat16), wm_ref[...].astype(jnp.bfloat16),
                      preferred_element_type=jnp.float32)
    o_ref[0] = jnp.maximum(out + b_ref[...], 0.0)


def _tc_interp(gathered, weights, W, b2):
    grid = (_B, _N_UP // _TILE_Q)
    return pl.pallas_call(
        _interp_body,
        grid=grid,
        in_specs=[
            pl.BlockSpec((_K, 1, _TILE_Q, _C), lambda bi, qi: (0, bi, qi, 0)),
            pl.BlockSpec((1, _TILE_Q, _K), lambda bi, qi: (bi, qi, 0)),
            pl.BlockSpec((_C, _DIM), lambda bi, qi: (0, 0)),
            pl.BlockSpec((1, _DIM), lambda bi, qi: (0, 0)),
        ],
        out_specs=pl.BlockSpec((1, _TILE_Q, _DIM), lambda bi, qi: (bi, qi, 0)),
        out_shape=jax.ShapeDtypeStruct((_B, _N_UP, _DIM), jnp.float32),
    )(gathered, weights, W, b2)


@jax.jit
def kernel(feature, pos, pos_up, W, b):
    posT = jnp.swapaxes(pos, 1, 2)
    b2 = b.reshape(1, _DIM)
    idx, weights = _tc_knn(pos_up, posT)
    flat_idx = jnp.transpose(idx, (2, 0, 1)).reshape(1, _K * _B * _N_UP)
    feature2 = feature.reshape(_B * _N, _C)
    gathered = _sc_gather(feature2, flat_idx)
    gathered = gathered.reshape(_K, _B, _N_UP, _C)
    return _tc_interp(gathered, weights, W, b2)


# speedup vs baseline: 2.0654x; 1.3145x over previous
import functools

import jax
import jax.numpy as jnp
from jax.experimental import pallas as pl
from jax.experimental.pallas import tpu as pltpu
from jax.experimental.pallas import tpu_sc as plsc

_B, _N, _N_UP, _C, _K, _DIM = 2, 4096, 8192, 128, 3, 128
_TILE_Q = 256
_GW = 128


_RG = 32
_NCHUNK = _N // 128


def _knn_body(q_ref, kT_ref, oi_ref, ow_ref, vals_ref, gidx_ref, qk_ref):
    bi = pl.program_id(0)
    kT = kT_ref[0]
    kx, ky, kz = kT[0:1, :], kT[1:2, :], kT[2:3, :]
    ksq = kx * kx + ky * ky + kz * kz
    q = q_ref[0]
    qk_ref[...] = jax.lax.dot(q.astype(jnp.bfloat16), kT.astype(jnp.bfloat16),
                              preferred_element_type=jnp.float32)

    lane = jax.lax.broadcasted_iota(jnp.int32, (_RG, 128), 1).astype(jnp.float32)
    big = jnp.full((_RG, 128), 3e38, jnp.float32)

    def row_group(rg, _):
        qg = q_ref[0, pl.ds(rg * _RG, _RG), :]
        qx, qy, qz = qg[:, 0:1], qg[:, 1:2], qg[:, 2:3]
        qsq = qx * qx + qy * qy + qz * qz
        qkg = qk_ref[pl.ds(rg * _RG, _RG), :]

        A1 = A2 = A3 = big
        I1 = I2 = I3 = jnp.zeros((_RG, 128), jnp.float32)
        for c in range(_NCHUNK):
            sl = slice(c * 128, (c + 1) * 128)
            v = (qsq + ksq[:, sl]) - 2.0 * qkg[:, sl]
            lt1 = v < A1
            lt2 = v < A2
            lt3 = v < A3
            fc = jnp.float32(c)
            A3 = jnp.minimum(jnp.maximum(v, A2), A3)
            I3 = jnp.where(lt2, I2, jnp.where(lt3, fc, I3))
            A2 = jnp.minimum(jnp.maximum(v, A1), A2)
            I2 = jnp.where(lt1, I1, jnp.where(lt2, fc, I2))
            A1 = jnp.minimum(v, A1)
            I1 = jnp.where(lt1, fc, I1)

        rows = pl.ds(rg * _RG, _RG)
        vals_ref[rows, 0:128] = A1
        vals_ref[rows, 128:256] = A2
        vals_ref[rows, 256:384] = A3
        gidx_ref[rows, 0:128] = I1 * 128.0 + lane
        gidx_ref[rows, 128:256] = I2 * 128.0 + lane
        gidx_ref[rows, 256:384] = I3 * 128.0 + lane

    for _rg in range(_TILE_Q // _RG):
        row_group(_rg, None)

    V = vals_ref[...]
    G = gidx_ref[...]
    idxs, ws = [], []
    for k in range(_K):
        m = jnp.min(V, axis=1, keepdims=True)
        isel = jnp.min(jnp.where(V == m, G, jnp.float32(_N)),
                       axis=1, keepdims=True)
        dist = jnp.sqrt(jnp.maximum(m, 1e-12))
        ws.append(1.0 / (dist + 1e-6))
        idxs.append(isel)
        if k < _K - 1:
            V = jnp.where((V == m) & (G == isel), jnp.float32(3e38), V)

    wsum = ws[0] + ws[1] + ws[2]
    ow_ref[0] = jnp.concatenate([w / wsum for w in ws], axis=1)
    oi_ref[0] = (jnp.concatenate(idxs, axis=1).astype(jnp.int32)
                 + bi * _N)


def _tc_knn(pos_up, posT):
    from jax.experimental.pallas import tpu as pltpu_mod
    grid = (_B, _N_UP // _TILE_Q)
    return pl.pallas_call(
        _knn_body,
        grid=grid,
        in_specs=[
            pl.BlockSpec((1, _TILE_Q, 3), lambda bi, qi: (bi, qi, 0)),
            pl.BlockSpec((1, 3, _N), lambda bi, qi: (bi, 0, 0)),
        ],
        out_specs=[
            pl.BlockSpec((1, _TILE_Q, _K), lambda bi, qi: (bi, qi, 0)),
            pl.BlockSpec((1, _TILE_Q, _K), lambda bi, qi: (bi, qi, 0)),
        ],
        out_shape=[
            jax.ShapeDtypeStruct((_B, _N_UP, _K), jnp.int32),
            jax.ShapeDtypeStruct((_B, _N_UP, _K), jnp.float32),
        ],
        scratch_shapes=[
            pltpu_mod.VMEM((_TILE_Q, 3 * 128), jnp.float32),
            pltpu_mod.VMEM((_TILE_Q, 3 * 128), jnp.float32),
            pltpu_mod.VMEM((_TILE_Q, _N), jnp.float32),
        ],
    )(pos_up, posT)


def _sc_gather(feature2, flat_idx):
    num_idx = flat_idx.shape[1]
    mesh = plsc.VectorSubcoreMesh(core_axis_name="core",
                                  subcore_axis_name="subcore")

    @pl.kernel(out_type=jax.ShapeDtypeStruct((num_idx, _C), jnp.float32),
               mesh=mesh)
    def kern(x_hbm, i_hbm, o_hbm):
        def body(i_vmem, o_vmem):
            pltpu.sync_copy(x_hbm.at[i_vmem.at[0]], o_vmem)

        pltpu.emit_pipeline(
            body,
            grid=(num_idx // _GW,),
            in_specs=[pl.BlockSpec((1, _GW), index_map=lambda i: (0, i))],
            out_specs=[pl.BlockSpec((_GW, _C), index_map=lambda i: (i, 0))],
            core_axis_name=("core", "subcore"),
            dimension_semantics=(pltpu.PARALLEL,),
        )(i_hbm, o_hbm)

    return kern(feature2, flat_idx)


def _interp_body(g_ref, w_ref, wm_ref, b_ref, o_ref):
    wn = w_ref[0]
    G = (wn[:, 0:1] * g_ref[0, 0]
         + wn[:, 1:2] * g_ref[1, 0]
         + wn[:, 2:3] * g_ref[2, 0])
    out = jax.lax.dot(G.astype(jnp.bfloat16), wm_ref[...].astype(jnp.bfloat16),
                      preferred_element_type=jnp.float32)
    o_ref[0] = jnp.maximum(out + b_ref[...], 0.0)


def _tc_interp(gathered, weights, W, b2):
    grid = (_B, _N_UP // _TILE_Q)
    return pl.pallas_call(
        _interp_body,
        grid=grid,
        in_specs=[
            pl.BlockSpec((_K, 1, _TILE_Q, _C), lambda bi, qi: (0, bi, qi, 0)),
            pl.BlockSpec((1, _TILE_Q, _K), lambda bi, qi: (bi, qi, 0)),
            pl.BlockSpec((_C, _DIM), lambda bi, qi: (0, 0)),
            pl.BlockSpec((1, _DIM), lambda bi, qi: (0, 0)),
        ],
        out_specs=pl.BlockSpec((1, _TILE_Q, _DIM), lambda bi, qi: (bi, qi, 0)),
        out_shape=jax.ShapeDtypeStruct((_B, _N_UP, _DIM), jnp.float32),
    )(gathered, weights, W, b2)


@jax.jit
def kernel(feature, pos, pos_up, W, b):
    posT = jnp.swapaxes(pos, 1, 2)
    b2 = b.reshape(1, _DIM)
    idx, weights = _tc_knn(pos_up, posT)
    flat_idx = jnp.transpose(idx, (2, 0, 1)).reshape(1, _K * _B * _N_UP)
    feature2 = feature.reshape(_B * _N, _C)
    gathered = _sc_gather(feature2, flat_idx)
    gathered = gathered.reshape(_K, _B, _N_UP, _C)
    return _tc_interp(gathered, weights, W, b2)


# speedup vs baseline: 2.1485x; 1.0402x over previous
import functools

import jax
import jax.numpy as jnp
from jax.experimental import pallas as pl
from jax.experimental.pallas import tpu as pltpu
from jax.experimental.pallas import tpu_sc as plsc

_B, _N, _N_UP, _C, _K, _DIM = 2, 4096, 8192, 128, 3, 128
_TILE_Q = 256
_RG = 32
_NCHUNK = _N // 128
_GW = 128


def _knn_body(q_ref, kT_ref, oi_ref, ow_ref, vals_ref, gidx_ref, qk_ref, *,
              base):
    kT = kT_ref[...]
    kx, ky, kz = kT[0:1, :], kT[1:2, :], kT[2:3, :]
    ksq = kx * kx + ky * ky + kz * kz
    q = q_ref[...]
    qk_ref[...] = jax.lax.dot(q.astype(jnp.bfloat16), kT.astype(jnp.bfloat16),
                              preferred_element_type=jnp.float32)

    lane = jax.lax.broadcasted_iota(jnp.int32, (_RG, 128), 1).astype(jnp.float32)
    big = jnp.full((_RG, 128), 3e38, jnp.float32)

    def row_group(rg):
        qg = q_ref[pl.ds(rg * _RG, _RG), :]
        qx, qy, qz = qg[:, 0:1], qg[:, 1:2], qg[:, 2:3]
        qsq = 0.25 * (qx * qx + qy * qy + qz * qz)
        qkg = qk_ref[pl.ds(rg * _RG, _RG), :]

        A1 = A2 = A3 = big
        I1 = I2 = I3 = jnp.zeros((_RG, 128), jnp.float32)
        for c in range(_NCHUNK):
            sl = slice(c * 128, (c + 1) * 128)
            v = (qsq + ksq[:, sl]) + qkg[:, sl]
            lt1 = v < A1
            lt2 = v < A2
            lt3 = v < A3
            fc = jnp.float32(c)
            A3 = jnp.minimum(jnp.maximum(v, A2), A3)
            I3 = jnp.where(lt2, I2, jnp.where(lt3, fc, I3))
            A2 = jnp.minimum(jnp.maximum(v, A1), A2)
            I2 = jnp.where(lt1, I1, jnp.where(lt2, fc, I2))
            A1 = jnp.minimum(v, A1)
            I1 = jnp.where(lt1, fc, I1)

        rows = pl.ds(rg * _RG, _RG)
        vals_ref[rows, 0:128] = A1
        vals_ref[rows, 128:256] = A2
        vals_ref[rows, 256:384] = A3
        gidx_ref[rows, 0:128] = I1 * 128.0 + lane
        gidx_ref[rows, 128:256] = I2 * 128.0 + lane
        gidx_ref[rows, 256:384] = I3 * 128.0 + lane

    for _rg in range(_TILE_Q // _RG):
        row_group(_rg)

    V = vals_ref[...]
    G = gidx_ref[...]
    idxs, ws = [], []
    for k in range(_K):
        m = jnp.min(V, axis=1, keepdims=True)
        isel = jnp.min(jnp.where(V == m, G, jnp.float32(_N)),
                       axis=1, keepdims=True)
        dist = jnp.sqrt(jnp.maximum(m, 1e-12))
        ws.append(1.0 / (dist + 1e-6))
        idxs.append(isel)
        if k < _K - 1:
            V = jnp.where((V == m) & (G == isel), jnp.float32(3e38), V)

    wsum = ws[0] + ws[1] + ws[2]
    ow_ref[...] = jnp.concatenate([w / wsum for w in ws], axis=1)
    oi_ref[...] = jnp.concatenate(idxs, axis=1).astype(jnp.int32) + base


def _tc_knn(pos_up_b, posT_b, base):
    grid = (_N_UP // _TILE_Q,)
    return pl.pallas_call(
        functools.partial(_knn_body, base=base),
        grid=grid,
        in_specs=[
            pl.BlockSpec((_TILE_Q, 3), lambda qi: (qi, 0)),
            pl.BlockSpec((3, _N), lambda qi: (0, 0)),
        ],
        out_specs=[
            pl.BlockSpec((_TILE_Q, _K), lambda qi: (qi, 0)),
            pl.BlockSpec((_TILE_Q, _K), lambda qi: (qi, 0)),
        ],
        out_shape=[
            jax.ShapeDtypeStruct((_N_UP, _K), jnp.int32),
            jax.ShapeDtypeStruct((_N_UP, _K), jnp.float32),
        ],
        scratch_shapes=[
            pltpu.VMEM((_TILE_Q, 3 * 128), jnp.float32),
            pltpu.VMEM((_TILE_Q, 3 * 128), jnp.float32),
            pltpu.VMEM((_TILE_Q, _N), jnp.float32),
        ],
    )(pos_up_b, posT_b)


def _sc_gather(feature2, flat_idx):
    num_idx = flat_idx.shape[1]
    mesh = plsc.VectorSubcoreMesh(core_axis_name="core",
                                  subcore_axis_name="subcore")

    @pl.kernel(out_type=jax.ShapeDtypeStruct((num_idx, _C), jnp.float32),
               mesh=mesh)
    def kern(x_hbm, i_hbm, o_hbm):
        def body(i_vmem, o_vmem):
            pltpu.sync_copy(x_hbm.at[i_vmem.at[0]], o_vmem)

        pltpu.emit_pipeline(
            body,
            grid=(num_idx // _GW,),
            in_specs=[pl.BlockSpec((1, _GW), index_map=lambda i: (0, i))],
            out_specs=[pl.BlockSpec((_GW, _C), index_map=lambda i: (i, 0))],
            core_axis_name=("core", "subcore"),
            dimension_semantics=(pltpu.PARALLEL,),
        )(i_hbm, o_hbm)

    return kern(feature2, flat_idx)


def _interp_body(g_ref, w_ref, wm_ref, b_ref, o_ref):
    wn = w_ref[...]
    G = (wn[:, 0:1] * g_ref[0]
         + wn[:, 1:2] * g_ref[1]
         + wn[:, 2:3] * g_ref[2])
    out = jax.lax.dot(G.astype(jnp.bfloat16), wm_ref[...].astype(jnp.bfloat16),
                      preferred_element_type=jnp.float32)
    o_ref[...] = jnp.maximum(out + b_ref[...], 0.0)


def _tc_interp(gathered_b, weights_b, W, b2):
    grid = (_N_UP // _TILE_Q,)
    return pl.pallas_call(
        _interp_body,
        grid=grid,
        in_specs=[
            pl.BlockSpec((_K, _TILE_Q, _C), lambda qi: (0, qi, 0)),
            pl.BlockSpec((_TILE_Q, _K), lambda qi: (qi, 0)),
            pl.BlockSpec((_C, _DIM), lambda qi: (0, 0)),
            pl.BlockSpec((1, _DIM), lambda qi: (0, 0)),
        ],
        out_specs=pl.BlockSpec((_TILE_Q, _DIM), lambda qi: (qi, 0)),
        out_shape=jax.ShapeDtypeStruct((_N_UP, _DIM), jnp.float32),
    )(gathered_b, weights_b, W, b2)


@jax.jit
def kernel(feature, pos, pos_up, W, b):
    posT = jnp.swapaxes(pos, 1, 2)
    qm2 = pos_up * jnp.float32(-2.0)
    b2 = b.reshape(1, _DIM)
    feature2 = feature.reshape(_B * _N, _C)
    outs = []
    for bi in range(_B):
        idx_b, w_b = _tc_knn(qm2[bi], posT[bi], bi * _N)
        flat_b = jnp.transpose(idx_b).reshape(1, _K * _N_UP)
        gathered = _sc_gather(feature2, flat_b)
        gathered = gathered.reshape(_K, _N_UP, _C)
        outs.append(_tc_interp(gathered, w_b, W, b2))
    return jnp.stack(outs)


# speedup vs baseline: 2.4236x; 1.1280x over previous
import functools

import jax
import jax.numpy as jnp
from jax.experimental import pallas as pl
from jax.experimental.pallas import tpu as pltpu
from jax.experimental.pallas import tpu_sc as plsc

_B, _N, _N_UP, _C, _K, _DIM = 2, 4096, 8192, 128, 3, 128
_TILE_Q = 512
_RG = 32
_NCHUNK = _N // 128
_GW = 128


def _knn_body(q_ref, kT_ref, oi_ref, ow_ref, vals_ref, gidx_ref, qk_ref, *,
              base):
    kT = kT_ref[...]
    kx, ky, kz = kT[0:1, :], kT[1:2, :], kT[2:3, :]
    ksq = kx * kx + ky * ky + kz * kz
    q = q_ref[...]
    qk_ref[...] = jax.lax.dot(q.astype(jnp.bfloat16), kT.astype(jnp.bfloat16),
                              preferred_element_type=jnp.float32)

    lane = jax.lax.broadcasted_iota(jnp.int32, (_RG, 128), 1).astype(jnp.float32)
    big = jnp.full((_RG, 128), 3e38, jnp.float32)

    def row_group(rg):
        qg = q_ref[pl.ds(rg * _RG, _RG), :]
        qx, qy, qz = qg[:, 0:1], qg[:, 1:2], qg[:, 2:3]
        qsq = 0.25 * (qx * qx + qy * qy + qz * qz)
        qkg = qk_ref[pl.ds(rg * _RG, _RG), :]

        A1 = A2 = A3 = big
        I1 = I2 = I3 = jnp.zeros((_RG, 128), jnp.float32)
        for c in range(_NCHUNK):
            sl = slice(c * 128, (c + 1) * 128)
            v = (qsq + ksq[:, sl]) + qkg[:, sl]
            lt1 = v < A1
            lt2 = v < A2
            lt3 = v < A3
            fc = jnp.float32(c)
            A3 = jnp.minimum(jnp.maximum(v, A2), A3)
            I3 = jnp.where(lt2, I2, jnp.where(lt3, fc, I3))
            A2 = jnp.minimum(jnp.maximum(v, A1), A2)
            I2 = jnp.where(lt1, I1, jnp.where(lt2, fc, I2))
            A1 = jnp.minimum(v, A1)
            I1 = jnp.where(lt1, fc, I1)

        rows = pl.ds(rg * _RG, _RG)
        vals_ref[rows, 0:128] = A1
        vals_ref[rows, 128:256] = A2
        vals_ref[rows, 256:384] = A3
        gidx_ref[rows, 0:128] = I1 * 128.0 + lane
        gidx_ref[rows, 128:256] = I2 * 128.0 + lane
        gidx_ref[rows, 256:384] = I3 * 128.0 + lane

    for _rg in range(_TILE_Q // _RG):
        row_group(_rg)

    V = vals_ref[...]
    G = gidx_ref[...]
    idxs, ws = [], []
    for k in range(_K):
        m = jnp.min(V, axis=1, keepdims=True)
        isel = jnp.min(jnp.where(V == m, G, jnp.float32(_N)),
                       axis=1, keepdims=True)
        dist = jnp.sqrt(jnp.maximum(m, 1e-12))
        ws.append(1.0 / (dist + 1e-6))
        idxs.append(isel)
        if k < _K - 1:
            V = jnp.where((V == m) & (G == isel), jnp.float32(3e38), V)

    wsum = ws[0] + ws[1] + ws[2]
    ow_ref[...] = jnp.concatenate([w / wsum for w in ws], axis=1)
    oi_ref[...] = jnp.concatenate(idxs, axis=1).astype(jnp.int32) + base


def _tc_knn(pos_up_b, posT_b, base):
    grid = (_N_UP // _TILE_Q,)
    return pl.pallas_call(
        functools.partial(_knn_body, base=base),
        grid=grid,
        in_specs=[
            pl.BlockSpec((_TILE_Q, 3), lambda qi: (qi, 0)),
            pl.BlockSpec((3, _N), lambda qi: (0, 0)),
        ],
        out_specs=[
            pl.BlockSpec((_TILE_Q, _K), lambda qi: (qi, 0)),
            pl.BlockSpec((_TILE_Q, _K), lambda qi: (qi, 0)),
        ],
        out_shape=[
            jax.ShapeDtypeStruct((_N_UP, _K), jnp.int32),
            jax.ShapeDtypeStruct((_N_UP, _K), jnp.float32),
        ],
        scratch_shapes=[
            pltpu.VMEM((_TILE_Q, 3 * 128), jnp.float32),
            pltpu.VMEM((_TILE_Q, 3 * 128), jnp.float32),
            pltpu.VMEM((_TILE_Q, _N), jnp.float32),
        ],
    )(pos_up_b, posT_b)


def _sc_gather(feature2, flat_idx):
    num_idx = flat_idx.shape[1]
    mesh = plsc.VectorSubcoreMesh(core_axis_name="core",
                                  subcore_axis_name="subcore")

    @pl.kernel(out_type=jax.ShapeDtypeStruct((num_idx, _C), jnp.float32),
               mesh=mesh)
    def kern(x_hbm, i_hbm, o_hbm):
        def body(i_vmem, o_vmem):
            pltpu.sync_copy(x_hbm.at[i_vmem.at[0]], o_vmem)

        pltpu.emit_pipeline(
            body,
            grid=(num_idx // _GW,),
            in_specs=[pl.BlockSpec((1, _GW), index_map=lambda i: (0, i))],
            out_specs=[pl.BlockSpec((_GW, _C), index_map=lambda i: (i, 0))],
            core_axis_name=("core", "subcore"),
            dimension_semantics=(pltpu.PARALLEL,),
        )(i_hbm, o_hbm)

    return kern(feature2, flat_idx)


def _interp_body(g_ref, w_ref, wm_ref, b_ref, o_ref):
    wn = w_ref[...]
    G = (wn[:, 0:1] * g_ref[0]
         + wn[:, 1:2] * g_ref[1]
         + wn[:, 2:3] * g_ref[2])
    out = jax.lax.dot(G.astype(jnp.bfloat16), wm_ref[...].astype(jnp.bfloat16),
                      preferred_element_type=jnp.float32)
    o_ref[...] = jnp.maximum(out + b_ref[...], 0.0)


def _tc_interp(gathered_b, weights_b, W, b2):
    grid = (_N_UP // _TILE_Q,)
    return pl.pallas_call(
        _interp_body,
        grid=grid,
        in_specs=[
            pl.BlockSpec((_K, _TILE_Q, _C), lambda qi: (0, qi, 0)),
            pl.BlockSpec((_TILE_Q, _K), lambda qi: (qi, 0)),
            pl.BlockSpec((_C, _DIM), lambda qi: (0, 0)),
            pl.BlockSpec((1, _DIM), lambda qi: (0, 0)),
        ],
        out_specs=pl.BlockSpec((_TILE_Q, _DIM), lambda qi: (qi, 0)),
        out_shape=jax.ShapeDtypeStruct((_N_UP, _DIM), jnp.float32),
    )(gathered_b, weights_b, W, b2)


@jax.jit
def kernel(feature, pos, pos_up, W, b):
    posT = jnp.swapaxes(pos, 1, 2)
    qm2 = pos_up * jnp.float32(-2.0)
    b2 = b.reshape(1, _DIM)
    feature2 = feature.reshape(_B * _N, _C)
    outs = []
    for bi in range(_B):
        idx_b, w_b = _tc_knn(qm2[bi], posT[bi], bi * _N)
        flat_b = jnp.transpose(idx_b).reshape(1, _K * _N_UP)
        gathered = _sc_gather(feature2, flat_b)
        gathered = gathered.reshape(_K, _N_UP, _C)
        outs.append(_tc_interp(gathered, w_b, W, b2))
    return jnp.stack(outs)


# speedup vs baseline: 2.4461x; 1.0093x over previous
import functools

import jax
import jax.numpy as jnp
from jax.experimental import pallas as pl
from jax.experimental.pallas import tpu as pltpu
from jax.experimental.pallas import tpu_sc as plsc

_B, _N, _N_UP, _C, _K, _DIM = 2, 4096, 8192, 128, 3, 128
_TILE_Q = 1024
_RG = 32
_NCHUNK = _N // 128
_GW = 128


def _knn_body(q_ref, kT_ref, oi_ref, ow_ref, vals_ref, gidx_ref, qk_ref, *,
              base):
    kT = kT_ref[...]
    kx, ky, kz = kT[0:1, :], kT[1:2, :], kT[2:3, :]
    ksq = kx * kx + ky * ky + kz * kz
    q = q_ref[...]
    qk_ref[...] = jax.lax.dot(q.astype(jnp.bfloat16), kT.astype(jnp.bfloat16),
                              preferred_element_type=jnp.float32)

    lane = jax.lax.broadcasted_iota(jnp.int32, (_RG, 128), 1).astype(jnp.float32)
    big = jnp.full((_RG, 128), 3e38, jnp.float32)

    def row_group(rg):
        qg = q_ref[pl.ds(rg * _RG, _RG), :]
        qx, qy, qz = qg[:, 0:1], qg[:, 1:2], qg[:, 2:3]
        qsq = 0.25 * (qx * qx + qy * qy + qz * qz)
        qkg = qk_ref[pl.ds(rg * _RG, _RG), :]

        A1 = A2 = A3 = big
        I1 = I2 = I3 = jnp.zeros((_RG, 128), jnp.float32)
        for c in range(_NCHUNK):
            sl = slice(c * 128, (c + 1) * 128)
            v = (qsq + ksq[:, sl]) + qkg[:, sl]
            lt1 = v < A1
            lt2 = v < A2
            lt3 = v < A3
            fc = jnp.float32(c)
            A3 = jnp.minimum(jnp.maximum(v, A2), A3)
            I3 = jnp.where(lt2, I2, jnp.where(lt3, fc, I3))
            A2 = jnp.minimum(jnp.maximum(v, A1), A2)
            I2 = jnp.where(lt1, I1, jnp.where(lt2, fc, I2))
            A1 = jnp.minimum(v, A1)
            I1 = jnp.where(lt1, fc, I1)

        rows = pl.ds(rg * _RG, _RG)
        vals_ref[rows, 0:128] = A1
        vals_ref[rows, 128:256] = A2
        vals_ref[rows, 256:384] = A3
        gidx_ref[rows, 0:128] = I1 * 128.0 + lane
        gidx_ref[rows, 128:256] = I2 * 128.0 + lane
        gidx_ref[rows, 256:384] = I3 * 128.0 + lane

    for _rg in range(_TILE_Q // _RG):
        row_group(_rg)

    V = vals_ref[...]
    G = gidx_ref[...]
    idxs, ws = [], []
    for k in range(_K):
        m = jnp.min(V, axis=1, keepdims=True)
        isel = jnp.min(jnp.where(V == m, G, jnp.float32(_N)),
                       axis=1, keepdims=True)
        dist = jnp.sqrt(jnp.maximum(m, 1e-12))
        ws.append(1.0 / (dist + 1e-6))
        idxs.append(isel)
        if k < _K - 1:
            V = jnp.where((V == m) & (G == isel), jnp.float32(3e38), V)

    wsum = ws[0] + ws[1] + ws[2]
    ow_ref[...] = jnp.concatenate([w / wsum for w in ws], axis=1)
    oi_ref[...] = jnp.concatenate(idxs, axis=1).astype(jnp.int32) + base


def _tc_knn(pos_up_b, posT_b, base):
    grid = (_N_UP // _TILE_Q,)
    return pl.pallas_call(
        functools.partial(_knn_body, base=base),
        grid=grid,
        in_specs=[
            pl.BlockSpec((_TILE_Q, 3), lambda qi: (qi, 0)),
            pl.BlockSpec((3, _N), lambda qi: (0, 0)),
        ],
        out_specs=[
            pl.BlockSpec((_TILE_Q, _K), lambda qi: (qi, 0)),
            pl.BlockSpec((_TILE_Q, _K), lambda qi: (qi, 0)),
        ],
        out_shape=[
            jax.ShapeDtypeStruct((_N_UP, _K), jnp.int32),
            jax.ShapeDtypeStruct((_N_UP, _K), jnp.float32),
        ],
        scratch_shapes=[
            pltpu.VMEM((_TILE_Q, 3 * 128), jnp.float32),
            pltpu.VMEM((_TILE_Q, 3 * 128), jnp.float32),
            pltpu.VMEM((_TILE_Q, _N), jnp.float32),
        ],
    )(pos_up_b, posT_b)


def _sc_gather(feature2, flat_idx):
    num_idx = flat_idx.shape[1]
    mesh = plsc.VectorSubcoreMesh(core_axis_name="core",
                                  subcore_axis_name="subcore")

    @pl.kernel(out_type=jax.ShapeDtypeStruct((num_idx, _C), jnp.float32),
               mesh=mesh)
    def kern(x_hbm, i_hbm, o_hbm):
        def body(i_vmem, o_vmem):
            pltpu.sync_copy(x_hbm.at[i_vmem.at[0]], o_vmem)

        pltpu.emit_pipeline(
            body,
            grid=(num_idx // _GW,),
            in_specs=[pl.BlockSpec((1, _GW), index_map=lambda i: (0, i))],
            out_specs=[pl.BlockSpec((_GW, _C), index_map=lambda i: (i, 0))],
            core_axis_name=("core", "subcore"),
            dimension_semantics=(pltpu.PARALLEL,),
        )(i_hbm, o_hbm)

    return kern(feature2, flat_idx)


def _interp_body(g_ref, w_ref, wm_ref, b_ref, o_ref):
    wn = w_ref[...]
    G = (wn[:, 0:1] * g_ref[0]
         + wn[:, 1:2] * g_ref[1]
         + wn[:, 2:3] * g_ref[2])
    out = jax.lax.dot(G.astype(jnp.bfloat16), wm_ref[...].astype(jnp.bfloat16),
                      preferred_element_type=jnp.float32)
    o_ref[...] = jnp.maximum(out + b_ref[...], 0.0)


def _tc_interp(gathered_b, weights_b, W, b2):
    grid = (_N_UP // _TILE_Q,)
    return pl.pallas_call(
        _interp_body,
        grid=grid,
        in_specs=[
            pl.BlockSpec((_K, _TILE_Q, _C), lambda qi: (0, qi, 0)),
            pl.BlockSpec((_TILE_Q, _K), lambda qi: (qi, 0)),
            pl.BlockSpec((_C, _DIM), lambda qi: (0, 0)),
            pl.BlockSpec((1, _DIM), lambda qi: (0, 0)),
        ],
        out_specs=pl.BlockSpec((_TILE_Q, _DIM), lambda qi: (qi, 0)),
        out_shape=jax.ShapeDtypeStruct((_N_UP, _DIM), jnp.float32),
    )(gathered_b, weights_b, W, b2)


@jax.jit
def kernel(feature, pos, pos_up, W, b):
    posT = jnp.swapaxes(pos, 1, 2)
    qm2 = pos_up * jnp.float32(-2.0)
    b2 = b.reshape(1, _DIM)
    feature2 = feature.reshape(_B * _N, _C)
    outs = []
    for bi in range(_B):
        idx_b, w_b = _tc_knn(qm2[bi], posT[bi], bi * _N)
        flat_b = jnp.transpose(idx_b).reshape(1, _K * _N_UP)
        gathered = _sc_gather(feature2, flat_b)
        gathered = gathered.reshape(_K, _N_UP, _C)
        outs.append(_tc_interp(gathered, w_b, W, b2))
    return jnp.stack(outs)


# speedup vs baseline: 2.5421x; 1.0392x over previous
import functools

import jax
import jax.numpy as jnp
from jax.experimental import pallas as pl
from jax.experimental.pallas import tpu as pltpu
from jax.experimental.pallas import tpu_sc as plsc

_B, _N, _N_UP, _C, _K, _DIM = 2, 4096, 8192, 128, 3, 128
_TILE_Q = 1024
_RG = 64
_NCHUNK = _N // 128
_GW = 128


def _knn_body(q_ref, kT_ref, oi_ref, ow_ref, vals_ref, gidx_ref, qk_ref, *,
              base):
    kT = kT_ref[...]
    kx, ky, kz = kT[0:1, :], kT[1:2, :], kT[2:3, :]
    ksq = kx * kx + ky * ky + kz * kz
    q = q_ref[...]
    qk_ref[...] = jax.lax.dot(q.astype(jnp.bfloat16), kT.astype(jnp.bfloat16),
                              preferred_element_type=jnp.float32)

    lane = jax.lax.broadcasted_iota(jnp.int32, (_RG, 128), 1).astype(jnp.float32)
    big = jnp.full((_RG, 128), 3e38, jnp.float32)

    def row_group(rg):
        qg = q_ref[pl.ds(rg * _RG, _RG), :]
        qx, qy, qz = qg[:, 0:1], qg[:, 1:2], qg[:, 2:3]
        qsq = 0.25 * (qx * qx + qy * qy + qz * qz)
        qkg = qk_ref[pl.ds(rg * _RG, _RG), :]

        A1 = A2 = A3 = big
        I1 = I2 = I3 = jnp.zeros((_RG, 128), jnp.float32)
        for c in range(_NCHUNK):
            sl = slice(c * 128, (c + 1) * 128)
            v = (qsq + ksq[:, sl]) + qkg[:, sl]
            lt1 = v < A1
            lt2 = v < A2
            lt3 = v < A3
            fc = jnp.float32(c)
            A3 = jnp.minimum(jnp.maximum(v, A2), A3)
            I3 = jnp.where(lt2, I2, jnp.where(lt3, fc, I3))
            A2 = jnp.minimum(jnp.maximum(v, A1), A2)
            I2 = jnp.where(lt1, I1, jnp.where(lt2, fc, I2))
            A1 = jnp.minimum(v, A1)
            I1 = jnp.where(lt1, fc, I1)

        rows = pl.ds(rg * _RG, _RG)
        vals_ref[rows, 0:128] = A1
        vals_ref[rows, 128:256] = A2
        vals_ref[rows, 256:384] = A3
        gidx_ref[rows, 0:128] = I1 * 128.0 + lane
        gidx_ref[rows, 128:256] = I2 * 128.0 + lane
        gidx_ref[rows, 256:384] = I3 * 128.0 + lane

    for _rg in range(_TILE_Q // _RG):
        row_group(_rg)

    V = vals_ref[...]
    G = gidx_ref[...]
    idxs, ws = [], []
    for k in range(_K):
        m = jnp.min(V, axis=1, keepdims=True)
        isel = jnp.min(jnp.where(V == m, G, jnp.float32(_N)),
                       axis=1, keepdims=True)
        dist = jnp.sqrt(jnp.maximum(m, 1e-12))
        ws.append(1.0 / (dist + 1e-6))
        idxs.append(isel)
        if k < _K - 1:
            V = jnp.where((V == m) & (G == isel), jnp.float32(3e38), V)

    wsum = ws[0] + ws[1] + ws[2]
    ow_ref[...] = jnp.concatenate([w / wsum for w in ws], axis=1)
    oi_ref[...] = jnp.concatenate(idxs, axis=1).astype(jnp.int32) + base


def _tc_knn(pos_up_b, posT_b, base):
    grid = (_N_UP // _TILE_Q,)
    return pl.pallas_call(
        functools.partial(_knn_body, base=base),
        grid=grid,
        in_specs=[
            pl.BlockSpec((_TILE_Q, 3), lambda qi: (qi, 0)),
            pl.BlockSpec((3, _N), lambda qi: (0, 0)),
        ],
        out_specs=[
            pl.BlockSpec((_TILE_Q, _K), lambda qi: (qi, 0)),
            pl.BlockSpec((_TILE_Q, _K), lambda qi: (qi, 0)),
        ],
        out_shape=[
            jax.ShapeDtypeStruct((_N_UP, _K), jnp.int32),
            jax.ShapeDtypeStruct((_N_UP, _K), jnp.float32),
        ],
        scratch_shapes=[
            pltpu.VMEM((_TILE_Q, 3 * 128), jnp.float32),
            pltpu.VMEM((_TILE_Q, 3 * 128), jnp.float32),
            pltpu.VMEM((_TILE_Q, _N), jnp.float32),
        ],
    )(pos_up_b, posT_b)


def _sc_gather(feature2, flat_idx):
    num_idx = flat_idx.shape[1]
    mesh = plsc.VectorSubcoreMesh(core_axis_name="core",
                                  subcore_axis_name="subcore")

    @pl.kernel(out_type=jax.ShapeDtypeStruct((num_idx, _C), jnp.float32),
               mesh=mesh)
    def kern(x_hbm, i_hbm, o_hbm):
        def body(i_vmem, o_vmem):
            pltpu.sync_copy(x_hbm.at[i_vmem.at[0]], o_vmem)

        pltpu.emit_pipeline(
            body,
            grid=(num_idx // _GW,),
            in_specs=[pl.BlockSpec((1, _GW), index_map=lambda i: (0, i))],
            out_specs=[pl.BlockSpec((_GW, _C), index_map=lambda i: (i, 0))],
            core_axis_name=("core", "subcore"),
            dimension_semantics=(pltpu.PARALLEL,),
        )(i_hbm, o_hbm)

    return kern(feature2, flat_idx)


def _interp_body(g_ref, w_ref, wm_ref, b_ref, o_ref):
    wn = w_ref[...]
    G = (wn[:, 0:1] * g_ref[0]
         + wn[:, 1:2] * g_ref[1]
         + wn[:, 2:3] * g_ref[2])
    out = jax.lax.dot(G.astype(jnp.bfloat16), wm_ref[...].astype(jnp.bfloat16),
                      preferred_element_type=jnp.float32)
    o_ref[...] = jnp.maximum(out + b_ref[...], 0.0)


def _tc_interp(gathered_b, weights_b, W, b2):
    grid = (_N_UP // _TILE_Q,)
    return pl.pallas_call(
        _interp_body,
        grid=grid,
        in_specs=[
            pl.BlockSpec((_K, _TILE_Q, _C), lambda qi: (0, qi, 0)),
            pl.BlockSpec((_TILE_Q, _K), lambda qi: (qi, 0)),
            pl.BlockSpec((_C, _DIM), lambda qi: (0, 0)),
            pl.BlockSpec((1, _DIM), lambda qi: (0, 0)),
        ],
        out_specs=pl.BlockSpec((_TILE_Q, _DIM), lambda qi: (qi, 0)),
        out_shape=jax.ShapeDtypeStruct((_N_UP, _DIM), jnp.float32),
    )(gathered_b, weights_b, W, b2)


@jax.jit
def kernel(feature, pos, pos_up, W, b):
    posT = jnp.swapaxes(pos, 1, 2)
    qm2 = pos_up * jnp.float32(-2.0)
    b2 = b.reshape(1, _DIM)
    feature2 = feature.reshape(_B * _N, _C)
    outs = []
    for bi in range(_B):
        idx_b, w_b = _tc_knn(qm2[bi], posT[bi], bi * _N)
        flat_b = jnp.transpose(idx_b).reshape(1, _K * _N_UP)
        gathered = _sc_gather(feature2, flat_b)
        gathered = gathered.reshape(_K, _N_UP, _C)
        outs.append(_tc_interp(gathered, w_b, W, b2))
    return jnp.stack(outs)
